# baseline (speedup 1.0000x reference)
import functools

import jax
import jax.numpy as jnp
from jax import lax
from jax.experimental import pallas as pl
from jax.experimental.pallas import tpu as pltpu
from jax.experimental.pallas import tpu_sc as plsc

_TAU = 5.0
_LAMBDA_MSE = 0.5

_S = 16384
_K = 200
_FLAT = _S * _K
_IDXW = 128
_ROWS = _FLAT // _IDXW
_NW = 32
_ROWS_PER_W = _ROWS // _NW
_G = 16
_STEPS = _ROWS_PER_W // _G


def _sc_gather(idx_flat, theta):
    mesh = plsc.VectorSubcoreMesh(core_axis_name="c", subcore_axis_name="s")

    @functools.partial(
        pl.kernel,
        mesh=mesh,
        out_type=jax.ShapeDtypeStruct((_FLAT,), jnp.float32),
        scratch_types=[
            pltpu.VMEM((_G * _IDXW,), jnp.int32),
            pltpu.VMEM((_G * _IDXW,), jnp.float32),
            pltpu.SemaphoreType.DMA,
        ],
    )
    def gather_kernel(idx_hbm, theta_hbm, out_hbm, idx_v, val_v, sem):
        nc = lax.axis_size("c")
        wid = lax.axis_index("s") * nc + lax.axis_index("c")
        base = wid * _ROWS_PER_W * _IDXW
        chunk = _G * _IDXW

        def step(g, carry):
            r0 = base + g * chunk
            pltpu.sync_copy(idx_hbm.at[pl.ds(r0, chunk)], idx_v)
            pltpu.async_copy(theta_hbm.at[idx_v], val_v, sem).wait()
            pltpu.sync_copy(val_v, out_hbm.at[pl.ds(r0, chunk)])
            return carry

        lax.fori_loop(0, _STEPS, step, 0)

    return gather_kernel(idx_flat, theta)


def _tc_body(a_ref, lens_ref, b_ref, t_ref, sc_ref, out_ref, acc_ref):
    i = pl.program_id(0)
    n = pl.num_programs(0)

    @pl.when(i == 0)
    def _init():
        acc_ref[0] = 0.0
        acc_ref[1] = 0.0
        acc_ref[2] = 0.0
        acc_ref[3] = 0.0

    t = t_ref[...] * _TAU
    bs = t.shape[0]
    kio = lax.broadcasted_iota(jnp.int32, (bs, _K), 1)
    mask = kio < lens_ref[...]
    maskf = mask.astype(jnp.float32)

    e = jnp.where(mask, jnp.exp(t), 0.0)
    rj = lax.broadcasted_iota(jnp.int32, (_K, _K), 0)
    ci = lax.broadcasted_iota(jnp.int32, (_K, _K), 1)
    tri = (rj >= ci).astype(jnp.float32)
    cumexp = jnp.dot(e, tri, preferred_element_type=jnp.float32)
    logc = jnp.log(cumexp + 1e-12)

    sum_t = jnp.sum(t * maskf)
    sum_lc = jnp.sum(logc * maskf)

    sc = sc_ref[...]
    w = jnp.maximum(1.0 / (1.0 + jnp.exp(-(sc - 0.5))), 0.1)
    wm = w * maskf
    pred = a_ref[0, 0] * t + b_ref[...]
    d = pred - sc * _TAU
    sum_se = jnp.sum(d * d * wm)
    sum_wm = jnp.sum(wm)

    acc_ref[0] += sum_t
    acc_ref[1] += sum_lc
    acc_ref[2] += sum_se
    acc_ref[3] += sum_wm

    @pl.when(i == n - 1)
    def _fin():
        nll = -(acc_ref[0] - acc_ref[1]) / float(_S)
        mse = acc_ref[2] / acc_ref[3]
        out_ref[0] = (1.0 - _LAMBDA_MSE) * nll + _LAMBDA_MSE * mse
        out_ref[1] = nll
        out_ref[2] = mse


def _tc_reduce(t, scores, lens, a, b_s):
    bs = 512
    grid = _S // bs
    out = pl.pallas_call(
        _tc_body,
        grid=(grid,),
        in_specs=[
            pl.BlockSpec((1, 1), lambda i: (0, 0), memory_space=pltpu.SMEM),
            pl.BlockSpec((bs, 1), lambda i: (i, 0)),
            pl.BlockSpec((bs, 1), lambda i: (i, 0)),
            pl.BlockSpec((bs, _K), lambda i: (i, 0)),
            pl.BlockSpec((bs, _K), lambda i: (i, 0)),
        ],
        out_specs=pl.BlockSpec(memory_space=pltpu.SMEM),
        out_shape=jax.ShapeDtypeStruct((3,), jnp.float32),
        scratch_shapes=[pltpu.SMEM((4,), jnp.float32)],
    )(
        jnp.asarray(a, jnp.float32).reshape(1, 1),
        lens.reshape(_S, 1),
        b_s.reshape(_S, 1),
        t,
        scores,
    )
    return out


def kernel(slates, scores, lens, theta, a, b_s):
    idx_flat = slates.reshape(_FLAT)
    gathered = _sc_gather(idx_flat, theta)
    t = gathered.reshape(_S, _K)
    out = _tc_reduce(t, scores, lens, a, b_s)
    return (out[0], out[1], out[2])


# speedup vs baseline: 1.1347x; 1.1347x over previous
import functools

import jax
import jax.numpy as jnp
from jax import lax
from jax.experimental import pallas as pl
from jax.experimental.pallas import tpu as pltpu
from jax.experimental.pallas import tpu_sc as plsc

_TAU = 5.0
_LAMBDA_MSE = 0.5

_S = 16384
_K = 200
_FLAT = _S * _K
_IDXW = 128
_ROWS = _FLAT // _IDXW
_NW = 32
_ROWS_PER_W = _ROWS // _NW
_G = 16
_STEPS = _ROWS_PER_W // _G


def _sc_gather(idx_flat, theta):
    mesh = plsc.VectorSubcoreMesh(core_axis_name="c", subcore_axis_name="s")

    @functools.partial(
        pl.kernel,
        mesh=mesh,
        out_type=jax.ShapeDtypeStruct((_FLAT,), jnp.float32),
        scratch_types=[
            pltpu.VMEM((_G * _IDXW,), jnp.int32),
            pltpu.VMEM((_G * _IDXW,), jnp.int32),
            pltpu.VMEM((_G * _IDXW,), jnp.float32),
            pltpu.VMEM((_G * _IDXW,), jnp.float32),
            pltpu.SemaphoreType.DMA,
            pltpu.SemaphoreType.DMA,
            pltpu.SemaphoreType.DMA,
            pltpu.SemaphoreType.DMA,
            pltpu.SemaphoreType.DMA,
            pltpu.SemaphoreType.DMA,
        ],
    )
    def gather_kernel(idx_hbm, theta_hbm, out_hbm, idx0, idx1, val0, val1,
                      si0, si1, sg0, sg1, sw0, sw1):
        nc = lax.axis_size("c")
        wid = lax.axis_index("s") * nc + lax.axis_index("c")
        base = wid * _ROWS_PER_W * _IDXW
        chunk = _G * _IDXW
        idx_v = (idx0, idx1)
        val_v = (val0, val1)
        semi = (si0, si1)
        semg = (sg0, sg1)
        semw = (sw0, sw1)

        def idx_start(g, p):
            pltpu.async_copy(
                idx_hbm.at[pl.ds(base + g * chunk, chunk)], idx_v[p], semi[p]
            )

        idx_start(0, 0)

        def phase(g, p):
            pltpu.make_async_copy(
                idx_hbm.at[pl.ds(0, chunk)], idx_v[p], semi[p]
            ).wait()
            @pl.when(g >= 2)
            def _():
                pltpu.make_async_copy(
                    val_v[p], out_hbm.at[pl.ds(0, chunk)], semw[p]
                ).wait()
            pltpu.async_copy(theta_hbm.at[idx_v[p]], val_v[p], semg[p])
            @pl.when(g + 1 < _STEPS)
            def _():
                pltpu.async_copy(
                    idx_hbm.at[pl.ds(base + (g + 1) * chunk, chunk)],
                    idx_v[1 - p],
                    semi[1 - p],
                )
            pltpu.make_async_copy(theta_hbm.at[idx_v[p]], val_v[p], semg[p]).wait()
            pltpu.async_copy(
                val_v[p], out_hbm.at[pl.ds(base + g * chunk, chunk)], semw[p]
            )

        def pair(g2, carry):
            phase(2 * g2, 0)
            phase(2 * g2 + 1, 1)
            return carry

        lax.fori_loop(0, _STEPS // 2, pair, 0)
        pltpu.make_async_copy(val0, out_hbm.at[pl.ds(0, chunk)], sw0).wait()
        pltpu.make_async_copy(val1, out_hbm.at[pl.ds(0, chunk)], sw1).wait()

    return gather_kernel(idx_flat, theta)


def _tc_body(a_ref, lens_ref, b_ref, t_ref, sc_ref, out_ref, acc_ref):
    i = pl.program_id(0)
    n = pl.num_programs(0)

    @pl.when(i == 0)
    def _init():
        acc_ref[0] = 0.0
        acc_ref[1] = 0.0
        acc_ref[2] = 0.0
        acc_ref[3] = 0.0

    t = t_ref[...] * _TAU
    bs = t.shape[0]
    kio = lax.broadcasted_iota(jnp.int32, (bs, _K), 1)
    mask = kio < lens_ref[...]
    maskf = mask.astype(jnp.float32)

    e = jnp.where(mask, jnp.exp(t), 0.0)
    rj = lax.broadcasted_iota(jnp.int32, (_K, _K), 0)
    ci = lax.broadcasted_iota(jnp.int32, (_K, _K), 1)
    tri = (rj >= ci).astype(jnp.float32)
    cumexp = jnp.dot(e, tri, preferred_element_type=jnp.float32)
    logc = jnp.log(cumexp + 1e-12)

    sum_t = jnp.sum(t * maskf)
    sum_lc = jnp.sum(logc * maskf)

    sc = sc_ref[...]
    w = jnp.maximum(1.0 / (1.0 + jnp.exp(-(sc - 0.5))), 0.1)
    wm = w * maskf
    pred = a_ref[0, 0] * t + b_ref[...]
    d = pred - sc * _TAU
    sum_se = jnp.sum(d * d * wm)
    sum_wm = jnp.sum(wm)

    acc_ref[0] += sum_t
    acc_ref[1] += sum_lc
    acc_ref[2] += sum_se
    acc_ref[3] += sum_wm

    @pl.when(i == n - 1)
    def _fin():
        nll = -(acc_ref[0] - acc_ref[1]) / float(_S)
        mse = acc_ref[2] / acc_ref[3]
        out_ref[0] = (1.0 - _LAMBDA_MSE) * nll + _LAMBDA_MSE * mse
        out_ref[1] = nll
        out_ref[2] = mse


def _tc_reduce(t, scores, lens, a, b_s):
    bs = 512
    grid = _S // bs
    out = pl.pallas_call(
        _tc_body,
        grid=(grid,),
        in_specs=[
            pl.BlockSpec((1, 1), lambda i: (0, 0), memory_space=pltpu.SMEM),
            pl.BlockSpec((bs, 1), lambda i: (i, 0)),
            pl.BlockSpec((bs, 1), lambda i: (i, 0)),
            pl.BlockSpec((bs, _K), lambda i: (i, 0)),
            pl.BlockSpec((bs, _K), lambda i: (i, 0)),
        ],
        out_specs=pl.BlockSpec(memory_space=pltpu.SMEM),
        out_shape=jax.ShapeDtypeStruct((3,), jnp.float32),
        scratch_shapes=[pltpu.SMEM((4,), jnp.float32)],
    )(
        jnp.asarray(a, jnp.float32).reshape(1, 1),
        lens.reshape(_S, 1),
        b_s.reshape(_S, 1),
        t,
        scores,
    )
    return out


def kernel(slates, scores, lens, theta, a, b_s):
    idx_flat = slates.reshape(_FLAT)
    gathered = _sc_gather(idx_flat, theta)
    t = gathered.reshape(_S, _K)
    out = _tc_reduce(t, scores, lens, a, b_s)
    return (out[0], out[1], out[2])


# speedup vs baseline: 1.2499x; 1.1015x over previous
import functools

import jax
import jax.numpy as jnp
from jax import lax
from jax.experimental import pallas as pl
from jax.experimental.pallas import tpu as pltpu
from jax.experimental.pallas import tpu_sc as plsc

_TAU = 5.0
_LAMBDA_MSE = 0.5

_S = 16384
_K = 200
_FLAT = _S * _K
_IDXW = 128
_ROWS = _FLAT // _IDXW
_NW = 32
_ROWS_PER_W = _ROWS // _NW
_G = 16
_STEPS = _ROWS_PER_W // _G


def _sc_gather(idx_flat, theta):
    mesh = plsc.VectorSubcoreMesh(core_axis_name="c", subcore_axis_name="s")

    @functools.partial(
        pl.kernel,
        mesh=mesh,
        out_type=jax.ShapeDtypeStruct((_FLAT,), jnp.float32),
        scratch_types=[
            pltpu.VMEM((_G * _IDXW,), jnp.int32),
            pltpu.VMEM((_G * _IDXW,), jnp.int32),
            pltpu.VMEM((_G * _IDXW,), jnp.float32),
            pltpu.VMEM((_G * _IDXW,), jnp.float32),
            pltpu.SemaphoreType.DMA,
            pltpu.SemaphoreType.DMA,
            pltpu.SemaphoreType.DMA,
            pltpu.SemaphoreType.DMA,
            pltpu.SemaphoreType.DMA,
            pltpu.SemaphoreType.DMA,
        ],
    )
    def gather_kernel(idx_hbm, theta_hbm, out_hbm, idx0, idx1, val0, val1,
                      si0, si1, sg0, sg1, sw0, sw1):
        nc = lax.axis_size("c")
        wid = lax.axis_index("s") * nc + lax.axis_index("c")
        base = wid * _ROWS_PER_W * _IDXW
        chunk = _G * _IDXW
        idx_v = (idx0, idx1)
        val_v = (val0, val1)
        semi = (si0, si1)
        semg = (sg0, sg1)
        semw = (sw0, sw1)

        def idx_start(g, p):
            pltpu.async_copy(
                idx_hbm.at[pl.ds(base + g * chunk, chunk)], idx_v[p], semi[p]
            )

        idx_start(0, 0)

        def phase(g, p):
            q = 1 - p
            pltpu.make_async_copy(
                idx_hbm.at[pl.ds(0, chunk)], idx_v[p], semi[p]
            ).wait()
            @pl.when(g >= 2)
            def _():
                pltpu.make_async_copy(
                    val_v[p], out_hbm.at[pl.ds(0, chunk)], semw[p]
                ).wait()
            pltpu.async_copy(theta_hbm.at[idx_v[p]], val_v[p], semg[p])
            @pl.when(g >= 1)
            def _():
                pltpu.make_async_copy(
                    theta_hbm.at[idx_v[q]], val_v[q], semg[q]
                ).wait()
                pltpu.async_copy(
                    val_v[q],
                    out_hbm.at[pl.ds(base + (g - 1) * chunk, chunk)],
                    semw[q],
                )
            @pl.when(g + 1 < _STEPS)
            def _():
                pltpu.async_copy(
                    idx_hbm.at[pl.ds(base + (g + 1) * chunk, chunk)],
                    idx_v[q],
                    semi[q],
                )

        def pair(g2, carry):
            phase(2 * g2, 0)
            phase(2 * g2 + 1, 1)
            return carry

        lax.fori_loop(0, _STEPS // 2, pair, 0)
        pltpu.make_async_copy(theta_hbm.at[idx1], val1, sg1).wait()
        pltpu.async_copy(
            val1, out_hbm.at[pl.ds(base + (_STEPS - 1) * chunk, chunk)], sw1
        )
        pltpu.make_async_copy(val0, out_hbm.at[pl.ds(0, chunk)], sw0).wait()
        pltpu.make_async_copy(val1, out_hbm.at[pl.ds(0, chunk)], sw1).wait()

    return gather_kernel(idx_flat, theta)


def _tc_body(a_ref, lens_ref, b_ref, t_ref, sc_ref, out_ref, acc_ref):
    i = pl.program_id(0)
    n = pl.num_programs(0)

    @pl.when(i == 0)
    def _init():
        acc_ref[0] = 0.0
        acc_ref[1] = 0.0
        acc_ref[2] = 0.0
        acc_ref[3] = 0.0

    t = t_ref[...] * _TAU
    bs = t.shape[0]
    kio = lax.broadcasted_iota(jnp.int32, (bs, _K), 1)
    mask = kio < lens_ref[...]
    maskf = mask.astype(jnp.float32)

    e = jnp.where(mask, jnp.exp(t), 0.0)
    rj = lax.broadcasted_iota(jnp.int32, (_K, _K), 0)
    ci = lax.broadcasted_iota(jnp.int32, (_K, _K), 1)
    tri = (rj >= ci).astype(jnp.float32)
    cumexp = jnp.dot(e, tri, preferred_element_type=jnp.float32)
    logc = jnp.log(cumexp + 1e-12)

    sum_t = jnp.sum(t * maskf)
    sum_lc = jnp.sum(logc * maskf)

    sc = sc_ref[...]
    w = jnp.maximum(1.0 / (1.0 + jnp.exp(-(sc - 0.5))), 0.1)
    wm = w * maskf
    pred = a_ref[0, 0] * t + b_ref[...]
    d = pred - sc * _TAU
    sum_se = jnp.sum(d * d * wm)
    sum_wm = jnp.sum(wm)

    acc_ref[0] += sum_t
    acc_ref[1] += sum_lc
    acc_ref[2] += sum_se
    acc_ref[3] += sum_wm

    @pl.when(i == n - 1)
    def _fin():
        nll = -(acc_ref[0] - acc_ref[1]) / float(_S)
        mse = acc_ref[2] / acc_ref[3]
        out_ref[0] = (1.0 - _LAMBDA_MSE) * nll + _LAMBDA_MSE * mse
        out_ref[1] = nll
        out_ref[2] = mse


def _tc_reduce(t, scores, lens, a, b_s):
    bs = 512
    grid = _S // bs
    out = pl.pallas_call(
        _tc_body,
        grid=(grid,),
        in_specs=[
            pl.BlockSpec((1, 1), lambda i: (0, 0), memory_space=pltpu.SMEM),
            pl.BlockSpec((bs, 1), lambda i: (i, 0)),
            pl.BlockSpec((bs, 1), lambda i: (i, 0)),
            pl.BlockSpec((bs, _K), lambda i: (i, 0)),
            pl.BlockSpec((bs, _K), lambda i: (i, 0)),
        ],
        out_specs=pl.BlockSpec(memory_space=pltpu.SMEM),
        out_shape=jax.ShapeDtypeStruct((3,), jnp.float32),
        scratch_shapes=[pltpu.SMEM((4,), jnp.float32)],
    )(
        jnp.asarray(a, jnp.float32).reshape(1, 1),
        lens.reshape(_S, 1),
        b_s.reshape(_S, 1),
        t,
        scores,
    )
    return out


def kernel(slates, scores, lens, theta, a, b_s):
    idx_flat = slates.reshape(_FLAT)
    gathered = _sc_gather(idx_flat, theta)
    t = gathered.reshape(_S, _K)
    out = _tc_reduce(t, scores, lens, a, b_s)
    return (out[0], out[1], out[2])


# speedup vs baseline: 1.2675x; 1.0141x over previous
import functools

import jax
import jax.numpy as jnp
from jax import lax
from jax.experimental import pallas as pl
from jax.experimental.pallas import tpu as pltpu
from jax.experimental.pallas import tpu_sc as plsc

_TAU = 5.0
_LAMBDA_MSE = 0.5

_S = 16384
_K = 200
_FLAT = _S * _K
_IDXW = 128
_ROWS = _FLAT // _IDXW
_NW = 32
_ROWS_PER_W = _ROWS // _NW
_G = 50
_STEPS = _ROWS_PER_W // _G


def _sc_gather(idx_flat, theta):
    mesh = plsc.VectorSubcoreMesh(core_axis_name="c", subcore_axis_name="s")

    @functools.partial(
        pl.kernel,
        mesh=mesh,
        out_type=jax.ShapeDtypeStruct((_FLAT,), jnp.float32),
        scratch_types=[
            pltpu.VMEM((_G * _IDXW,), jnp.int32),
            pltpu.VMEM((_G * _IDXW,), jnp.int32),
            pltpu.VMEM((_G * _IDXW,), jnp.float32),
            pltpu.VMEM((_G * _IDXW,), jnp.float32),
            pltpu.SemaphoreType.DMA,
            pltpu.SemaphoreType.DMA,
            pltpu.SemaphoreType.DMA,
            pltpu.SemaphoreType.DMA,
            pltpu.SemaphoreType.DMA,
            pltpu.SemaphoreType.DMA,
        ],
    )
    def gather_kernel(idx_hbm, theta_hbm, out_hbm, idx0, idx1, val0, val1,
                      si0, si1, sg0, sg1, sw0, sw1):
        nc = lax.axis_size("c")
        wid = lax.axis_index("s") * nc + lax.axis_index("c")
        base = wid * _ROWS_PER_W * _IDXW
        chunk = _G * _IDXW
        idx_v = (idx0, idx1)
        val_v = (val0, val1)
        semi = (si0, si1)
        semg = (sg0, sg1)
        semw = (sw0, sw1)

        def idx_start(g, p):
            pltpu.async_copy(
                idx_hbm.at[pl.ds(base + g * chunk, chunk)], idx_v[p], semi[p]
            )

        idx_start(0, 0)

        def phase(g, p):
            q = 1 - p
            pltpu.make_async_copy(
                idx_hbm.at[pl.ds(0, chunk)], idx_v[p], semi[p]
            ).wait()
            @pl.when(g >= 2)
            def _():
                pltpu.make_async_copy(
                    val_v[p], out_hbm.at[pl.ds(0, chunk)], semw[p]
                ).wait()
            pltpu.async_copy(theta_hbm.at[idx_v[p]], val_v[p], semg[p])
            @pl.when(g >= 1)
            def _():
                pltpu.make_async_copy(
                    theta_hbm.at[idx_v[q]], val_v[q], semg[q]
                ).wait()
                pltpu.async_copy(
                    val_v[q],
                    out_hbm.at[pl.ds(base + (g - 1) * chunk, chunk)],
                    semw[q],
                )
            @pl.when(g + 1 < _STEPS)
            def _():
                pltpu.async_copy(
                    idx_hbm.at[pl.ds(base + (g + 1) * chunk, chunk)],
                    idx_v[q],
                    semi[q],
                )

        def pair(g2, carry):
            phase(2 * g2, 0)
            phase(2 * g2 + 1, 1)
            return carry

        lax.fori_loop(0, _STEPS // 2, pair, 0)
        pltpu.make_async_copy(theta_hbm.at[idx1], val1, sg1).wait()
        pltpu.async_copy(
            val1, out_hbm.at[pl.ds(base + (_STEPS - 1) * chunk, chunk)], sw1
        )
        pltpu.make_async_copy(val0, out_hbm.at[pl.ds(0, chunk)], sw0).wait()
        pltpu.make_async_copy(val1, out_hbm.at[pl.ds(0, chunk)], sw1).wait()

    return gather_kernel(idx_flat, theta)


def _tc_body(a_ref, lens_ref, b_ref, t_ref, sc_ref, out_ref, acc_ref):
    i = pl.program_id(0)
    n = pl.num_programs(0)

    @pl.when(i == 0)
    def _init():
        acc_ref[0] = 0.0
        acc_ref[1] = 0.0
        acc_ref[2] = 0.0
        acc_ref[3] = 0.0

    t = t_ref[...] * _TAU
    bs = t.shape[0]
    kio = lax.broadcasted_iota(jnp.int32, (bs, _K), 1)
    mask = kio < lens_ref[...]
    maskf = mask.astype(jnp.float32)

    e = jnp.where(mask, jnp.exp(t), 0.0)
    rj = lax.broadcasted_iota(jnp.int32, (_K, _K), 0)
    ci = lax.broadcasted_iota(jnp.int32, (_K, _K), 1)
    tri = (rj >= ci).astype(jnp.float32)
    cumexp = jnp.dot(e, tri, preferred_element_type=jnp.float32)
    logc = jnp.log(cumexp + 1e-12)

    sum_t = jnp.sum(t * maskf)
    sum_lc = jnp.sum(logc * maskf)

    sc = sc_ref[...]
    w = jnp.maximum(1.0 / (1.0 + jnp.exp(-(sc - 0.5))), 0.1)
    wm = w * maskf
    pred = a_ref[0, 0] * t + b_ref[...]
    d = pred - sc * _TAU
    sum_se = jnp.sum(d * d * wm)
    sum_wm = jnp.sum(wm)

    acc_ref[0] += sum_t
    acc_ref[1] += sum_lc
    acc_ref[2] += sum_se
    acc_ref[3] += sum_wm

    @pl.when(i == n - 1)
    def _fin():
        nll = -(acc_ref[0] - acc_ref[1]) / float(_S)
        mse = acc_ref[2] / acc_ref[3]
        out_ref[0] = (1.0 - _LAMBDA_MSE) * nll + _LAMBDA_MSE * mse
        out_ref[1] = nll
        out_ref[2] = mse


def _tc_reduce(t, scores, lens, a, b_s):
    bs = 512
    grid = _S // bs
    out = pl.pallas_call(
        _tc_body,
        grid=(grid,),
        in_specs=[
            pl.BlockSpec((1, 1), lambda i: (0, 0), memory_space=pltpu.SMEM),
            pl.BlockSpec((bs, 1), lambda i: (i, 0)),
            pl.BlockSpec((bs, 1), lambda i: (i, 0)),
            pl.BlockSpec((bs, _K), lambda i: (i, 0)),
            pl.BlockSpec((bs, _K), lambda i: (i, 0)),
        ],
        out_specs=pl.BlockSpec(memory_space=pltpu.SMEM),
        out_shape=jax.ShapeDtypeStruct((3,), jnp.float32),
        scratch_shapes=[pltpu.SMEM((4,), jnp.float32)],
    )(
        jnp.asarray(a, jnp.float32).reshape(1, 1),
        lens.reshape(_S, 1),
        b_s.reshape(_S, 1),
        t,
        scores,
    )
    return out


def kernel(slates, scores, lens, theta, a, b_s):
    idx_flat = slates.reshape(_FLAT)
    gathered = _sc_gather(idx_flat, theta)
    t = gathered.reshape(_S, _K)
    out = _tc_reduce(t, scores, lens, a, b_s)
    return (out[0], out[1], out[2])


# speedup vs baseline: 1.3485x; 1.0639x over previous
import functools

import jax
import jax.numpy as jnp
from jax import lax
from jax.experimental import pallas as pl
from jax.experimental.pallas import tpu as pltpu
from jax.experimental.pallas import tpu_sc as plsc

_TAU = 5.0
_LAMBDA_MSE = 0.5

_S = 16384
_K = 200
_NW = 32
_CHUNK = 6400


def _sc_gather(idx_flat, theta):
    n = idx_flat.shape[0]
    per_w = n // _NW
    steps = per_w // _CHUNK
    assert per_w % _CHUNK == 0 and steps % 2 == 0
    mesh = plsc.VectorSubcoreMesh(core_axis_name="c", subcore_axis_name="s")

    @functools.partial(
        pl.kernel,
        mesh=mesh,
        out_type=jax.ShapeDtypeStruct((n,), jnp.float32),
        scratch_types=[
            pltpu.VMEM((_CHUNK,), jnp.int32),
            pltpu.VMEM((_CHUNK,), jnp.int32),
            pltpu.VMEM((_CHUNK,), jnp.float32),
            pltpu.VMEM((_CHUNK,), jnp.float32),
            pltpu.SemaphoreType.DMA,
            pltpu.SemaphoreType.DMA,
            pltpu.SemaphoreType.DMA,
            pltpu.SemaphoreType.DMA,
            pltpu.SemaphoreType.DMA,
            pltpu.SemaphoreType.DMA,
        ],
    )
    def gather_kernel(idx_hbm, theta_hbm, out_hbm, idx0, idx1, val0, val1,
                      si0, si1, sg0, sg1, sw0, sw1):
        nc = lax.axis_size("c")
        wid = lax.axis_index("s") * nc + lax.axis_index("c")
        base = wid * per_w
        idx_v = (idx0, idx1)
        val_v = (val0, val1)
        semi = (si0, si1)
        semg = (sg0, sg1)
        semw = (sw0, sw1)

        def idx_start(g, p):
            pltpu.async_copy(
                idx_hbm.at[pl.ds(base + g * _CHUNK, _CHUNK)], idx_v[p], semi[p]
            )

        idx_start(0, 0)

        def phase(g, p):
            q = 1 - p
            pltpu.make_async_copy(
                idx_hbm.at[pl.ds(0, _CHUNK)], idx_v[p], semi[p]
            ).wait()
            @pl.when(g >= 2)
            def _():
                pltpu.make_async_copy(
                    val_v[p], out_hbm.at[pl.ds(0, _CHUNK)], semw[p]
                ).wait()
            pltpu.async_copy(theta_hbm.at[idx_v[p]], val_v[p], semg[p])
            @pl.when(g >= 1)
            def _():
                pltpu.make_async_copy(
                    theta_hbm.at[idx_v[q]], val_v[q], semg[q]
                ).wait()
                pltpu.async_copy(
                    val_v[q],
                    out_hbm.at[pl.ds(base + (g - 1) * _CHUNK, _CHUNK)],
                    semw[q],
                )
            @pl.when(g + 1 < steps)
            def _():
                idx_start(g + 1, q)

        def pair(g2, carry):
            phase(2 * g2, 0)
            phase(2 * g2 + 1, 1)
            return carry

        lax.fori_loop(0, steps // 2, pair, 0)
        pltpu.make_async_copy(theta_hbm.at[idx1], val1, sg1).wait()
        pltpu.async_copy(
            val1, out_hbm.at[pl.ds(base + (steps - 1) * _CHUNK, _CHUNK)], sw1
        )
        pltpu.make_async_copy(val0, out_hbm.at[pl.ds(0, _CHUNK)], sw0).wait()
        pltpu.make_async_copy(val1, out_hbm.at[pl.ds(0, _CHUNK)], sw1).wait()

    return gather_kernel(idx_flat, theta)


def _make_tc_body(final):
    def tc_body(a_ref, prev_ref, lens_ref, b_ref, t_ref, sc_ref, out_ref,
                acc_ref):
        i = pl.program_id(0)
        n = pl.num_programs(0)

        @pl.when(i == 0)
        def _init():
            acc_ref[0] = prev_ref[0]
            acc_ref[1] = prev_ref[1]
            acc_ref[2] = prev_ref[2]
            acc_ref[3] = prev_ref[3]

        t = t_ref[...] * _TAU
        bs = t.shape[0]
        kio = lax.broadcasted_iota(jnp.int32, (bs, _K), 1)
        mask = kio < lens_ref[...]
        maskf = mask.astype(jnp.float32)

        e = jnp.where(mask, jnp.exp(t), 0.0)
        rj = lax.broadcasted_iota(jnp.int32, (_K, _K), 0)
        ci = lax.broadcasted_iota(jnp.int32, (_K, _K), 1)
        tri = (rj >= ci).astype(jnp.float32)
        cumexp = jnp.dot(e, tri, preferred_element_type=jnp.float32)
        logc = jnp.log(cumexp + 1e-12)

        sum_t = jnp.sum(t * maskf)
        sum_lc = jnp.sum(logc * maskf)

        sc = sc_ref[...]
        w = jnp.maximum(1.0 / (1.0 + jnp.exp(-(sc - 0.5))), 0.1)
        wm = w * maskf
        pred = a_ref[0, 0] * t + b_ref[...]
        d = pred - sc * _TAU
        sum_se = jnp.sum(d * d * wm)
        sum_wm = jnp.sum(wm)

        acc_ref[0] += sum_t
        acc_ref[1] += sum_lc
        acc_ref[2] += sum_se
        acc_ref[3] += sum_wm

        if final:
            @pl.when(i == n - 1)
            def _fin():
                nll = -(acc_ref[0] - acc_ref[1]) / float(_S)
                mse = acc_ref[2] / acc_ref[3]
                out_ref[0] = (1.0 - _LAMBDA_MSE) * nll + _LAMBDA_MSE * mse
                out_ref[1] = nll
                out_ref[2] = mse
        else:
            @pl.when(i == n - 1)
            def _fin():
                out_ref[0] = acc_ref[0]
                out_ref[1] = acc_ref[1]
                out_ref[2] = acc_ref[2]
                out_ref[3] = acc_ref[3]

    return tc_body


def _tc_reduce(t, scores, lens, a, b_s, prev, final):
    ns = t.shape[0]
    bs = 512
    grid = ns // bs
    out = pl.pallas_call(
        _make_tc_body(final),
        grid=(grid,),
        in_specs=[
            pl.BlockSpec((1, 1), lambda i: (0, 0), memory_space=pltpu.SMEM),
            pl.BlockSpec(memory_space=pltpu.SMEM),
            pl.BlockSpec((bs, 1), lambda i: (i, 0)),
            pl.BlockSpec((bs, 1), lambda i: (i, 0)),
            pl.BlockSpec((bs, _K), lambda i: (i, 0)),
            pl.BlockSpec((bs, _K), lambda i: (i, 0)),
        ],
        out_specs=pl.BlockSpec(memory_space=pltpu.SMEM),
        out_shape=jax.ShapeDtypeStruct((3 if final else 4,), jnp.float32),
        scratch_shapes=[pltpu.SMEM((4,), jnp.float32)],
    )(
        jnp.asarray(a, jnp.float32).reshape(1, 1),
        prev,
        lens.reshape(ns, 1),
        b_s.reshape(ns, 1),
        t,
        scores,
    )
    return out


def kernel(slates, scores, lens, theta, a, b_s):
    h = _S // 2
    g1 = _sc_gather(slates[:h].reshape(-1), theta)
    g2 = _sc_gather(slates[h:].reshape(-1), theta)
    zeros4 = jnp.zeros((4,), jnp.float32)
    p1 = _tc_reduce(
        g1.reshape(h, _K), scores[:h], lens[:h], a, b_s[:h], zeros4, False
    )
    out = _tc_reduce(
        g2.reshape(h, _K), scores[h:], lens[h:], a, b_s[h:], p1, True
    )
    return (out[0], out[1], out[2])


# speedup vs baseline: 1.4781x; 1.0961x over previous
import functools

import jax
import jax.numpy as jnp
from jax import lax
from jax.experimental import pallas as pl
from jax.experimental.pallas import tpu as pltpu
from jax.experimental.pallas import tpu_sc as plsc

_TAU = 5.0
_LAMBDA_MSE = 0.5

_S = 16384
_K = 200
_NW = 32
_CHUNK = 6400


def _sc_gather(idx_flat, theta):
    n = idx_flat.shape[0]
    per_w = n // _NW
    steps = per_w // _CHUNK
    assert per_w % _CHUNK == 0 and steps % 2 == 0
    mesh = plsc.VectorSubcoreMesh(core_axis_name="c", subcore_axis_name="s")

    @functools.partial(
        pl.kernel,
        mesh=mesh,
        out_type=jax.ShapeDtypeStruct((n,), jnp.float32),
        scratch_types=[
            pltpu.VMEM((_CHUNK,), jnp.int32),
            pltpu.VMEM((_CHUNK,), jnp.int32),
            pltpu.VMEM((_CHUNK,), jnp.float32),
            pltpu.VMEM((_CHUNK,), jnp.float32),
            pltpu.SemaphoreType.DMA,
            pltpu.SemaphoreType.DMA,
            pltpu.SemaphoreType.DMA,
            pltpu.SemaphoreType.DMA,
            pltpu.SemaphoreType.DMA,
            pltpu.SemaphoreType.DMA,
        ],
    )
    def gather_kernel(idx_hbm, theta_hbm, out_hbm, idx0, idx1, val0, val1,
                      si0, si1, sg0, sg1, sw0, sw1):
        nc = lax.axis_size("c")
        wid = lax.axis_index("s") * nc + lax.axis_index("c")
        base = wid * per_w
        idx_v = (idx0, idx1)
        val_v = (val0, val1)
        semi = (si0, si1)
        semg = (sg0, sg1)
        semw = (sw0, sw1)

        def idx_start(g, p):
            pltpu.async_copy(
                idx_hbm.at[pl.ds(base + g * _CHUNK, _CHUNK)], idx_v[p], semi[p]
            )

        idx_start(0, 0)

        def phase(g, p):
            q = 1 - p
            pltpu.make_async_copy(
                idx_hbm.at[pl.ds(0, _CHUNK)], idx_v[p], semi[p]
            ).wait()
            @pl.when(g >= 2)
            def _():
                pltpu.make_async_copy(
                    val_v[p], out_hbm.at[pl.ds(0, _CHUNK)], semw[p]
                ).wait()
            pltpu.async_copy(theta_hbm.at[idx_v[p]], val_v[p], semg[p])
            @pl.when(g >= 1)
            def _():
                pltpu.make_async_copy(
                    theta_hbm.at[idx_v[q]], val_v[q], semg[q]
                ).wait()
                pltpu.async_copy(
                    val_v[q],
                    out_hbm.at[pl.ds(base + (g - 1) * _CHUNK, _CHUNK)],
                    semw[q],
                )
            @pl.when(g + 1 < steps)
            def _():
                idx_start(g + 1, q)

        def pair(g2, carry):
            phase(2 * g2, 0)
            phase(2 * g2 + 1, 1)
            return carry

        lax.fori_loop(0, steps // 2, pair, 0)
        pltpu.make_async_copy(theta_hbm.at[idx1], val1, sg1).wait()
        pltpu.async_copy(
            val1, out_hbm.at[pl.ds(base + (steps - 1) * _CHUNK, _CHUNK)], sw1
        )
        pltpu.make_async_copy(val0, out_hbm.at[pl.ds(0, _CHUNK)], sw0).wait()
        pltpu.make_async_copy(val1, out_hbm.at[pl.ds(0, _CHUNK)], sw1).wait()

    return gather_kernel(idx_flat, theta)


def _make_tc_body(final):
    def tc_body(a_ref, prev_ref, lens_ref, b_ref, t_ref, sc_ref, out_ref,
                acc_ref):
        i = pl.program_id(0)
        n = pl.num_programs(0)

        @pl.when(i == 0)
        def _init():
            acc_ref[0] = prev_ref[0]
            acc_ref[1] = prev_ref[1]
            acc_ref[2] = prev_ref[2]
            acc_ref[3] = prev_ref[3]

        t = t_ref[...] * _TAU
        bs = t.shape[0]
        kio = lax.broadcasted_iota(jnp.int32, (bs, _K), 1)
        mask = kio < lens_ref[...]
        maskf = mask.astype(jnp.float32)

        e = jnp.where(mask, jnp.exp(t), 0.0)
        rj = lax.broadcasted_iota(jnp.int32, (_K, _K), 0)
        ci = lax.broadcasted_iota(jnp.int32, (_K, _K), 1)
        tri = (rj >= ci).astype(jnp.float32)
        cumexp = jnp.dot(e, tri, preferred_element_type=jnp.float32)
        logc = jnp.log(cumexp + 1e-12)

        sum_t = jnp.sum(t * maskf)
        sum_lc = jnp.sum(logc * maskf)

        sc = sc_ref[...]
        w = jnp.maximum(1.0 / (1.0 + jnp.exp(-(sc - 0.5))), 0.1)
        wm = w * maskf
        pred = a_ref[0, 0] * t + b_ref[...]
        d = pred - sc * _TAU
        sum_se = jnp.sum(d * d * wm)
        sum_wm = jnp.sum(wm)

        acc_ref[0] += sum_t
        acc_ref[1] += sum_lc
        acc_ref[2] += sum_se
        acc_ref[3] += sum_wm

        if final:
            @pl.when(i == n - 1)
            def _fin():
                nll = -(acc_ref[0] - acc_ref[1]) / float(_S)
                mse = acc_ref[2] / acc_ref[3]
                out_ref[0] = (1.0 - _LAMBDA_MSE) * nll + _LAMBDA_MSE * mse
                out_ref[1] = nll
                out_ref[2] = mse
        else:
            @pl.when(i == n - 1)
            def _fin():
                out_ref[0] = acc_ref[0]
                out_ref[1] = acc_ref[1]
                out_ref[2] = acc_ref[2]
                out_ref[3] = acc_ref[3]

    return tc_body


def _tc_reduce(t, scores, lens2d, a2d, b2d, prev, block0, final):
    ns = t.shape[0]
    bs = 512
    grid = ns // bs
    out = pl.pallas_call(
        _make_tc_body(final),
        grid=(grid,),
        in_specs=[
            pl.BlockSpec((1, 1), lambda i: (0, 0), memory_space=pltpu.SMEM),
            pl.BlockSpec(memory_space=pltpu.SMEM),
            pl.BlockSpec((bs, 1), lambda i: (i + block0, 0)),
            pl.BlockSpec((bs, 1), lambda i: (i + block0, 0)),
            pl.BlockSpec((bs, _K), lambda i: (i, 0)),
            pl.BlockSpec((bs, _K), lambda i: (i + block0, 0)),
        ],
        out_specs=pl.BlockSpec(memory_space=pltpu.SMEM),
        out_shape=jax.ShapeDtypeStruct((3 if final else 4,), jnp.float32),
        scratch_shapes=[pltpu.SMEM((4,), jnp.float32)],
    )(
        a2d,
        prev,
        lens2d,
        b2d,
        t,
        scores,
    )
    return out


_NSPLIT = 4


def kernel(slates, scores, lens, theta, a, b_s):
    h = _S // _NSPLIT
    nblk = h // 512
    a2d = jnp.asarray(a, jnp.float32).reshape(1, 1)
    lens2d = lens.reshape(_S, 1)
    b2d = b_s.reshape(_S, 1)
    gs = [
        _sc_gather(slates[i * h:(i + 1) * h].reshape(-1), theta)
        for i in range(_NSPLIT)
    ]
    acc = jnp.zeros((4,), jnp.float32)
    for i in range(_NSPLIT):
        acc = _tc_reduce(
            gs[i].reshape(h, _K), scores, lens2d, a2d, b2d, acc,
            i * nblk, i == _NSPLIT - 1
        )
    return (acc[0], acc[1], acc[2])


# speedup vs baseline: 1.5460x; 1.0459x over previous
import functools

import jax
import jax.numpy as jnp
from jax import lax
from jax.experimental import pallas as pl
from jax.experimental.pallas import tpu as pltpu
from jax.experimental.pallas import tpu_sc as plsc

_TAU = 5.0
_LAMBDA_MSE = 0.5

_S = 16384
_K = 200
_NW = 32
_CHUNK = 6400


def _sc_gather(idx_flat, theta):
    n = idx_flat.shape[0]
    per_w = n // _NW
    chunk = _CHUNK
    while per_w % chunk or (per_w // chunk) % 2:
        chunk //= 2
    steps = per_w // chunk
    mesh = plsc.VectorSubcoreMesh(core_axis_name="c", subcore_axis_name="s")

    @functools.partial(
        pl.kernel,
        mesh=mesh,
        out_type=jax.ShapeDtypeStruct((n,), jnp.float32),
        scratch_types=[
            pltpu.VMEM((chunk,), jnp.int32),
            pltpu.VMEM((chunk,), jnp.int32),
            pltpu.VMEM((chunk,), jnp.float32),
            pltpu.VMEM((chunk,), jnp.float32),
            pltpu.SemaphoreType.DMA,
            pltpu.SemaphoreType.DMA,
            pltpu.SemaphoreType.DMA,
            pltpu.SemaphoreType.DMA,
            pltpu.SemaphoreType.DMA,
            pltpu.SemaphoreType.DMA,
        ],
    )
    def gather_kernel(idx_hbm, theta_hbm, out_hbm, idx0, idx1, val0, val1,
                      si0, si1, sg0, sg1, sw0, sw1):
        nc = lax.axis_size("c")
        wid = lax.axis_index("s") * nc + lax.axis_index("c")
        base = wid * per_w
        idx_v = (idx0, idx1)
        val_v = (val0, val1)
        semi = (si0, si1)
        semg = (sg0, sg1)
        semw = (sw0, sw1)

        def idx_start(g, p):
            pltpu.async_copy(
                idx_hbm.at[pl.ds(base + g * chunk, chunk)], idx_v[p], semi[p]
            )

        idx_start(0, 0)

        def phase(g, p):
            q = 1 - p
            pltpu.make_async_copy(
                idx_hbm.at[pl.ds(0, chunk)], idx_v[p], semi[p]
            ).wait()
            @pl.when(g >= 2)
            def _():
                pltpu.make_async_copy(
                    val_v[p], out_hbm.at[pl.ds(0, chunk)], semw[p]
                ).wait()
            pltpu.async_copy(theta_hbm.at[idx_v[p]], val_v[p], semg[p])
            @pl.when(g >= 1)
            def _():
                pltpu.make_async_copy(
                    theta_hbm.at[idx_v[q]], val_v[q], semg[q]
                ).wait()
                pltpu.async_copy(
                    val_v[q],
                    out_hbm.at[pl.ds(base + (g - 1) * chunk, chunk)],
                    semw[q],
                )
            @pl.when(g + 1 < steps)
            def _():
                idx_start(g + 1, q)

        def pair(g2, carry):
            phase(2 * g2, 0)
            phase(2 * g2 + 1, 1)
            return carry

        lax.fori_loop(0, steps // 2, pair, 0)
        pltpu.make_async_copy(theta_hbm.at[idx1], val1, sg1).wait()
        pltpu.async_copy(
            val1, out_hbm.at[pl.ds(base + (steps - 1) * chunk, chunk)], sw1
        )
        pltpu.make_async_copy(val0, out_hbm.at[pl.ds(0, chunk)], sw0).wait()
        pltpu.make_async_copy(val1, out_hbm.at[pl.ds(0, chunk)], sw1).wait()

    return gather_kernel(idx_flat, theta)


def _make_tc_body(final):
    def tc_body(a_ref, prev_ref, lens_ref, b_ref, t_ref, sc_ref, out_ref,
                acc_ref):
        i = pl.program_id(0)
        n = pl.num_programs(0)

        @pl.when(i == 0)
        def _init():
            acc_ref[0] = prev_ref[0]
            acc_ref[1] = prev_ref[1]
            acc_ref[2] = prev_ref[2]
            acc_ref[3] = prev_ref[3]

        t = t_ref[...] * _TAU
        bs = t.shape[0]
        kio = lax.broadcasted_iota(jnp.int32, (bs, _K), 1)
        mask = kio < lens_ref[...]
        maskf = mask.astype(jnp.float32)

        e = jnp.where(mask, jnp.exp(t), 0.0)
        rj = lax.broadcasted_iota(jnp.int32, (_K, _K), 0)
        ci = lax.broadcasted_iota(jnp.int32, (_K, _K), 1)
        tri = (rj >= ci).astype(jnp.float32)
        cumexp = jnp.dot(e, tri, preferred_element_type=jnp.float32)
        logc = jnp.log(cumexp + 1e-12)

        sum_t = jnp.sum(t * maskf)
        sum_lc = jnp.sum(logc * maskf)

        sc = sc_ref[...]
        w = jnp.maximum(1.0 / (1.0 + jnp.exp(-(sc - 0.5))), 0.1)
        wm = w * maskf
        pred = a_ref[0, 0] * t + b_ref[...]
        d = pred - sc * _TAU
        sum_se = jnp.sum(d * d * wm)
        sum_wm = jnp.sum(wm)

        acc_ref[0] += sum_t
        acc_ref[1] += sum_lc
        acc_ref[2] += sum_se
        acc_ref[3] += sum_wm

        if final:
            @pl.when(i == n - 1)
            def _fin():
                nll = -(acc_ref[0] - acc_ref[1]) / float(_S)
                mse = acc_ref[2] / acc_ref[3]
                out_ref[0] = (1.0 - _LAMBDA_MSE) * nll + _LAMBDA_MSE * mse
                out_ref[1] = nll
                out_ref[2] = mse
        else:
            @pl.when(i == n - 1)
            def _fin():
                out_ref[0] = acc_ref[0]
                out_ref[1] = acc_ref[1]
                out_ref[2] = acc_ref[2]
                out_ref[3] = acc_ref[3]

    return tc_body


def _tc_reduce(t, scores, lens2d, a2d, b2d, prev, block0, final):
    ns = t.shape[0]
    bs = 512
    grid = ns // bs
    out = pl.pallas_call(
        _make_tc_body(final),
        grid=(grid,),
        in_specs=[
            pl.BlockSpec((1, 1), lambda i: (0, 0), memory_space=pltpu.SMEM),
            pl.BlockSpec(memory_space=pltpu.SMEM),
            pl.BlockSpec((bs, 1), lambda i: (i + block0, 0)),
            pl.BlockSpec((bs, 1), lambda i: (i + block0, 0)),
            pl.BlockSpec((bs, _K), lambda i: (i, 0)),
            pl.BlockSpec((bs, _K), lambda i: (i + block0, 0)),
        ],
        out_specs=pl.BlockSpec(memory_space=pltpu.SMEM),
        out_shape=jax.ShapeDtypeStruct((3 if final else 4,), jnp.float32),
        scratch_shapes=[pltpu.SMEM((4,), jnp.float32)],
    )(
        a2d,
        prev,
        lens2d,
        b2d,
        t,
        scores,
    )
    return out


_SPLITS = (2048, 4096, 4096, 4096, 2048)


def kernel(slates, scores, lens, theta, a, b_s):
    a2d = jnp.asarray(a, jnp.float32).reshape(1, 1)
    lens2d = lens.reshape(_S, 1)
    b2d = b_s.reshape(_S, 1)
    offs = [0]
    for ns in _SPLITS:
        offs.append(offs[-1] + ns)
    gs = [
        _sc_gather(slates[offs[i]:offs[i + 1]].reshape(-1), theta)
        for i in range(len(_SPLITS))
    ]
    acc = jnp.zeros((4,), jnp.float32)
    for i, ns in enumerate(_SPLITS):
        acc = _tc_reduce(
            gs[i].reshape(ns, _K), scores, lens2d, a2d, b2d, acc,
            offs[i] // 512, i == len(_SPLITS) - 1
        )
    return (acc[0], acc[1], acc[2])
